# Initial kernel scaffold; baseline (speedup 1.0000x reference)
import jax
import jax.numpy as jnp
from jax.experimental import pallas as pl


def kernel(features, edge_index, W1, b1, g1, beta1, W2, b2, g2, beta2, W3, b3):
    raise NotImplementedError("write your pallas kernel here")


# speedup vs baseline: 8.7733x; 8.7733x over previous
import functools

import jax
import jax.numpy as jnp
from jax import lax
from jax.experimental import pallas as pl
from jax.experimental.pallas import tpu as pltpu
from jax.experimental.pallas import tpu_sc as plsc

N, E, D, H, O = 10000, 320000, 128, 128, 2
NC, NS = 2, 16
NW = NC * NS
WIN = 128
NPAD = 16
NROWS = 10112
EPW = -(-(-(-E // NW)) // WIN) * WIN
NWIN = EPW // WIN
EP = EPW * NW
DEG_ROWS = 2 * NROWS
_DEG_RPS = DEG_ROWS // NS
_AGG_RPS = NROWS // NS

_mesh = plsc.VectorSubcoreMesh(core_axis_name="c", subcore_axis_name="s")


@functools.partial(
    pl.kernel,
    mesh=_mesh,
    out_type=jax.ShapeDtypeStruct((NC, DEG_ROWS, 16), jnp.float32),
    compiler_params=pltpu.CompilerParams(use_tc_tiling_on_sc=False),
    scratch_types=[
        pltpu.VMEM((NWIN, WIN), jnp.int32),
        pltpu.VMEM((NWIN, WIN), jnp.int32),
        pltpu.VMEM((WIN, 16), jnp.float32),
        pltpu.VMEM_SHARED((DEG_ROWS, 16), jnp.float32),
    ],
)
def _deg_kernel(sidx_hbm, didx_hbm, ones_hbm, zeros_hbm, out_hbm,
                sidx_v, didx_v, ones_v, acc_sh):
    c = lax.axis_index("c")
    s = lax.axis_index("s")
    wid = c * NS + s
    r0 = s * _DEG_RPS
    pltpu.sync_copy(zeros_hbm, acc_sh.at[pl.ds(r0, _DEG_RPS)])
    pltpu.sync_copy(ones_hbm, ones_v)
    pltpu.sync_copy(sidx_hbm.at[wid], sidx_v)
    pltpu.sync_copy(didx_hbm.at[wid], didx_v)
    plsc.subcore_barrier()

    @pl.loop(0, NWIN)
    def _(j):
        pltpu.sync_copy(ones_v, acc_sh.at[sidx_v.at[j]], add=True)
        pltpu.sync_copy(ones_v, acc_sh.at[didx_v.at[j]], add=True)

    plsc.subcore_barrier()
    pltpu.sync_copy(acc_sh.at[pl.ds(r0, _DEG_RPS)],
                    out_hbm.at[c, pl.ds(r0, _DEG_RPS)])


def _make_agg_kernel(width):
    @functools.partial(
        pl.kernel,
        mesh=_mesh,
        out_type=jax.ShapeDtypeStruct((NC, NROWS, width), jnp.float32),
        compiler_params=pltpu.CompilerParams(use_tc_tiling_on_sc=False),
        scratch_types=[
            pltpu.VMEM((NWIN, WIN), jnp.int32),
            pltpu.VMEM((NWIN, WIN), jnp.int32),
            pltpu.VMEM((WIN, width), jnp.float32),
            pltpu.VMEM_SHARED((NROWS, width), jnp.float32),
        ],
    )
    def _agg_kernel(y_hbm, gidx_hbm, sidx_hbm, zeros_hbm, out_hbm,
                    gidx_v, sidx_v, rows_v, acc_sh):
        c = lax.axis_index("c")
        s = lax.axis_index("s")
        wid = c * NS + s
        r0 = s * _AGG_RPS
        pltpu.sync_copy(zeros_hbm, acc_sh.at[pl.ds(r0, _AGG_RPS)])
        pltpu.sync_copy(gidx_hbm.at[wid], gidx_v)
        pltpu.sync_copy(sidx_hbm.at[wid], sidx_v)
        plsc.subcore_barrier()

        @pl.loop(0, NWIN)
        def _(j):
            pltpu.sync_copy(y_hbm.at[gidx_v.at[j]], rows_v)
            pltpu.sync_copy(rows_v, acc_sh.at[sidx_v.at[j]], add=True)

        plsc.subcore_barrier()
        pltpu.sync_copy(acc_sh.at[pl.ds(r0, _AGG_RPS)],
                        out_hbm.at[c, pl.ds(r0, _AGG_RPS)])

    return _agg_kernel


_W3COLS = 16
_agg128 = _make_agg_kernel(D)
_agg3 = _agg128 if _W3COLS == D else _make_agg_kernel(_W3COLS)


def _prep_body(degp_ref, feat_ref, ns_ref, nd_ref, y1_ref):
    od = degp_ref[0, :N, 0:1] + degp_ref[1, :N, 0:1]
    idg = (degp_ref[0, NROWS:NROWS + N, 0:1]
           + degp_ref[1, NROWS:NROWS + N, 0:1])
    ns = jnp.where(od > 0, lax.rsqrt(od), 0.0)
    nd = jnp.where(idg > 0, lax.rsqrt(idg), 0.0)
    ns_ref[...] = ns
    nd_ref[...] = nd
    y1_ref[...] = feat_ref[...] * ns


def _mid_body(p_ref, nd_ref, W_ref, b_ref, g_ref, be_ref, ns_ref, o_ref):
    agg = p_ref[0, :N, :] + p_ref[1, :N, :]
    t = jnp.dot(agg * nd_ref[...], W_ref[...],
                preferred_element_type=jnp.float32) + b_ref[...]
    mu = jnp.mean(t, -1, keepdims=True)
    var = jnp.mean((t - mu) ** 2, -1, keepdims=True)
    x = (t - mu) / jnp.sqrt(var + 1e-5) * g_ref[...] + be_ref[...]
    x = jnp.where(x > 0, x, 0.01 * x)
    o_ref[...] = x * ns_ref[...]


def _mid3_body(p_ref, nd_ref, W_ref, b_ref, g_ref, be_ref, ns_ref, W3_ref,
               o_ref):
    agg = p_ref[0, :N, :] + p_ref[1, :N, :]
    t = jnp.dot(agg * nd_ref[...], W_ref[...],
                preferred_element_type=jnp.float32) + b_ref[...]
    mu = jnp.mean(t, -1, keepdims=True)
    var = jnp.mean((t - mu) ** 2, -1, keepdims=True)
    x = (t - mu) / jnp.sqrt(var + 1e-5) * g_ref[...] + be_ref[...]
    x = jnp.where(x > 0, x, 0.01 * x)
    o_ref[...] = jnp.dot(x * ns_ref[...], W3_ref[...],
                         preferred_element_type=jnp.float32)


def _fin_body(p_ref, nd_ref, b3_ref, o_ref):
    a = p_ref[0, :N, :O] + p_ref[1, :N, :O]
    o_ref[...] = a * nd_ref[...] + b3_ref[...]


def kernel(features, edge_index, W1, b1, g1, beta1, W2, b2, g2, beta2, W3, b3):
    src = edge_index[0]
    dst = edge_index[1]

    pad = EP - E
    k = jnp.arange(pad, dtype=jnp.int32)
    kp = k % NPAD
    trash = N + kp
    src_g = jnp.concatenate([src, kp]).reshape(NW, NWIN, WIN)
    dst_s = jnp.concatenate([dst, trash]).reshape(NW, NWIN, WIN)
    src_d = jnp.concatenate([src, trash]).reshape(NW, NWIN, WIN)
    dst_d = jnp.concatenate([dst + NROWS, trash + NROWS]).reshape(NW, NWIN, WIN)

    ones16 = jnp.ones((WIN, 16), jnp.float32)
    zdeg = jnp.zeros((_DEG_RPS, 16), jnp.float32)
    zagg = jnp.zeros((_AGG_RPS, D), jnp.float32)
    zagg16 = jnp.zeros((_AGG_RPS, _W3COLS), jnp.float32)

    degp = _deg_kernel(src_d, dst_d, ones16, zdeg)

    ns, nd, y1 = pl.pallas_call(
        _prep_body,
        out_shape=(jax.ShapeDtypeStruct((N, 1), jnp.float32),
                   jax.ShapeDtypeStruct((N, 1), jnp.float32),
                   jax.ShapeDtypeStruct((N, D), jnp.float32)),
    )(degp, features)

    b1r, g1r, be1r = b1.reshape(1, H), g1.reshape(1, H), beta1.reshape(1, H)
    b2r, g2r, be2r = b2.reshape(1, H), g2.reshape(1, H), beta2.reshape(1, H)
    W3p = jnp.pad(W3, ((0, 0), (0, _W3COLS - O)))
    b3r = b3.reshape(1, O)

    p1 = _agg128(y1, src_g, dst_s, zagg)
    y2 = pl.pallas_call(
        _mid_body,
        out_shape=jax.ShapeDtypeStruct((N, D), jnp.float32),
    )(p1, nd, W1, b1r, g1r, be1r, ns)

    p2 = _agg128(y2, src_g, dst_s, zagg)
    z3 = pl.pallas_call(
        _mid3_body,
        out_shape=jax.ShapeDtypeStruct((N, _W3COLS), jnp.float32),
    )(p2, nd, W2, b2r, g2r, be2r, ns, W3p)

    p3 = _agg3(z3, src_g, dst_s, zagg16)
    out = pl.pallas_call(
        _fin_body,
        out_shape=jax.ShapeDtypeStruct((N, O), jnp.float32),
    )(p3, nd, b3r)
    return out


# speedup vs baseline: 9.9630x; 1.1356x over previous
import functools

import jax
import jax.numpy as jnp
from jax import lax
from jax.experimental import pallas as pl
from jax.experimental.pallas import tpu as pltpu
from jax.experimental.pallas import tpu_sc as plsc

N, E, D, H, O = 10000, 320000, 128, 128, 2
NC, NS = 2, 16
NW = NC * NS
WIN = 128
NPAD = 16
NROWS = 10112
NWIN = 2 * (-(-(-(-E // NW)) // (2 * WIN)))
EPW = NWIN * WIN
EP = EPW * NW
DEG_ROWS = 2 * NROWS
_DEG_RPS = DEG_ROWS // NS
_AGG_RPS = NROWS // NS

_mesh = plsc.VectorSubcoreMesh(core_axis_name="c", subcore_axis_name="s")


@functools.partial(
    pl.kernel,
    mesh=_mesh,
    out_type=jax.ShapeDtypeStruct((NC, DEG_ROWS, 16), jnp.float32),
    compiler_params=pltpu.CompilerParams(use_tc_tiling_on_sc=False),
    scratch_types=[
        pltpu.VMEM((NWIN, WIN), jnp.int32),
        pltpu.VMEM((NWIN, WIN), jnp.int32),
        pltpu.VMEM((WIN, 16), jnp.float32),
        pltpu.VMEM_SHARED((DEG_ROWS, 16), jnp.float32),
    ],
)
def _deg_kernel(sidx_hbm, didx_hbm, ones_hbm, zeros_hbm, out_hbm,
                sidx_v, didx_v, ones_v, acc_sh):
    c = lax.axis_index("c")
    s = lax.axis_index("s")
    wid = c * NS + s
    r0 = s * _DEG_RPS
    pltpu.sync_copy(zeros_hbm, acc_sh.at[pl.ds(r0, _DEG_RPS)])
    pltpu.sync_copy(ones_hbm, ones_v)
    pltpu.sync_copy(sidx_hbm.at[wid], sidx_v)
    pltpu.sync_copy(didx_hbm.at[wid], didx_v)
    plsc.subcore_barrier()

    @pl.loop(0, NWIN)
    def _(j):
        pltpu.sync_copy(ones_v, acc_sh.at[sidx_v.at[j]], add=True)
        pltpu.sync_copy(ones_v, acc_sh.at[didx_v.at[j]], add=True)

    plsc.subcore_barrier()
    pltpu.sync_copy(acc_sh.at[pl.ds(r0, _DEG_RPS)],
                    out_hbm.at[c, pl.ds(r0, _DEG_RPS)])


def _make_agg_kernel(width):
    @functools.partial(
        pl.kernel,
        mesh=_mesh,
        out_type=jax.ShapeDtypeStruct((NC, NROWS, width), jnp.float32),
        compiler_params=pltpu.CompilerParams(use_tc_tiling_on_sc=False),
        scratch_types=[
            pltpu.VMEM((2, WIN), jnp.int32),
            pltpu.VMEM((2, WIN), jnp.int32),
            pltpu.VMEM((WIN, width), jnp.float32),
            pltpu.VMEM((WIN, width), jnp.float32),
            pltpu.SemaphoreType.DMA,
            pltpu.SemaphoreType.DMA,
            pltpu.SemaphoreType.DMA,
            pltpu.SemaphoreType.DMA,
            pltpu.VMEM_SHARED((NROWS, width), jnp.float32),
        ],
    )
    def _agg_kernel(y_hbm, gidx_hbm, sidx_hbm, zeros_hbm, out_hbm,
                    gidx_v, sidx_v, rows0, rows1, semi0, semi1, semg0, semg1,
                    acc_sh):
        c = lax.axis_index("c")
        s = lax.axis_index("s")
        wid = c * NS + s
        r0 = s * _AGG_RPS
        pltpu.sync_copy(zeros_hbm, acc_sh.at[pl.ds(r0, _AGG_RPS)])

        def idx_start(j, b, sem):
            pltpu.make_async_copy(gidx_hbm.at[wid, j], gidx_v.at[b],
                                  sem).start()
            pltpu.make_async_copy(sidx_hbm.at[wid, j], sidx_v.at[b],
                                  sem).start()

        def idx_wait(j, b, sem):
            pltpu.make_async_copy(gidx_hbm.at[wid, j], gidx_v.at[b],
                                  sem).wait()
            pltpu.make_async_copy(sidx_hbm.at[wid, j], sidx_v.at[b],
                                  sem).wait()

        plsc.subcore_barrier()

        idx_start(0, 0, semi0)
        idx_wait(0, 0, semi0)
        pltpu.make_async_copy(y_hbm.at[gidx_v.at[0]], rows0, semg0).start()
        idx_start(1, 1, semi1)

        @pl.loop(0, NWIN, step=2)
        def _(j):
            pltpu.make_async_copy(y_hbm.at[gidx_v.at[0]], rows0,
                                  semg0).wait()
            idx_wait(j + 1, 1, semi1)
            pltpu.make_async_copy(y_hbm.at[gidx_v.at[1]], rows1,
                                  semg1).start()
            pltpu.sync_copy(rows0, acc_sh.at[sidx_v.at[0]], add=True)

            @pl.when(j + 2 < NWIN)
            def _():
                idx_start(j + 2, 0, semi0)

            pltpu.make_async_copy(y_hbm.at[gidx_v.at[1]], rows1,
                                  semg1).wait()

            @pl.when(j + 2 < NWIN)
            def _():
                idx_wait(j + 2, 0, semi0)
                pltpu.make_async_copy(y_hbm.at[gidx_v.at[0]], rows0,
                                      semg0).start()

            pltpu.sync_copy(rows1, acc_sh.at[sidx_v.at[1]], add=True)

            @pl.when(j + 3 < NWIN)
            def _():
                idx_start(j + 3, 1, semi1)

        plsc.subcore_barrier()
        pltpu.sync_copy(acc_sh.at[pl.ds(r0, _AGG_RPS)],
                        out_hbm.at[c, pl.ds(r0, _AGG_RPS)])

    return _agg_kernel


_W3COLS = 16
_agg128 = _make_agg_kernel(D)
_agg3 = _agg128 if _W3COLS == D else _make_agg_kernel(_W3COLS)


def _prep_body(degp_ref, feat_ref, ns_ref, nd_ref, y1_ref):
    od = degp_ref[0, :N, 0:1] + degp_ref[1, :N, 0:1]
    idg = (degp_ref[0, NROWS:NROWS + N, 0:1]
           + degp_ref[1, NROWS:NROWS + N, 0:1])
    ns = jnp.where(od > 0, lax.rsqrt(od), 0.0)
    nd = jnp.where(idg > 0, lax.rsqrt(idg), 0.0)
    ns_ref[...] = ns
    nd_ref[...] = nd
    y1_ref[...] = feat_ref[...] * ns


def _mid_body(p_ref, nd_ref, W_ref, b_ref, g_ref, be_ref, ns_ref, o_ref):
    agg = p_ref[0, :N, :] + p_ref[1, :N, :]
    t = jnp.dot(agg * nd_ref[...], W_ref[...],
                preferred_element_type=jnp.float32) + b_ref[...]
    mu = jnp.mean(t, -1, keepdims=True)
    var = jnp.mean((t - mu) ** 2, -1, keepdims=True)
    x = (t - mu) / jnp.sqrt(var + 1e-5) * g_ref[...] + be_ref[...]
    x = jnp.where(x > 0, x, 0.01 * x)
    o_ref[...] = x * ns_ref[...]


def _mid3_body(p_ref, nd_ref, W_ref, b_ref, g_ref, be_ref, ns_ref, W3_ref,
               o_ref):
    agg = p_ref[0, :N, :] + p_ref[1, :N, :]
    t = jnp.dot(agg * nd_ref[...], W_ref[...],
                preferred_element_type=jnp.float32) + b_ref[...]
    mu = jnp.mean(t, -1, keepdims=True)
    var = jnp.mean((t - mu) ** 2, -1, keepdims=True)
    x = (t - mu) / jnp.sqrt(var + 1e-5) * g_ref[...] + be_ref[...]
    x = jnp.where(x > 0, x, 0.01 * x)
    o_ref[...] = jnp.dot(x * ns_ref[...], W3_ref[...],
                         preferred_element_type=jnp.float32)


def _fin_body(p_ref, nd_ref, b3_ref, o_ref):
    a = p_ref[0, :N, :O] + p_ref[1, :N, :O]
    o_ref[...] = a * nd_ref[...] + b3_ref[...]


def kernel(features, edge_index, W1, b1, g1, beta1, W2, b2, g2, beta2, W3, b3):
    src = edge_index[0]
    dst = edge_index[1]

    pad = EP - E
    k = jnp.arange(pad, dtype=jnp.int32)
    kp = k % NPAD
    trash = N + kp
    src_g = jnp.concatenate([src, kp]).reshape(NW, NWIN, WIN)
    dst_s = jnp.concatenate([dst, trash]).reshape(NW, NWIN, WIN)
    src_d = jnp.concatenate([src, trash]).reshape(NW, NWIN, WIN)
    dst_d = jnp.concatenate([dst + NROWS, trash + NROWS]).reshape(NW, NWIN, WIN)

    ones16 = jnp.ones((WIN, 16), jnp.float32)
    zdeg = jnp.zeros((_DEG_RPS, 16), jnp.float32)
    zagg = jnp.zeros((_AGG_RPS, D), jnp.float32)
    zagg16 = jnp.zeros((_AGG_RPS, _W3COLS), jnp.float32)

    degp = _deg_kernel(src_d, dst_d, ones16, zdeg)

    ns, nd, y1 = pl.pallas_call(
        _prep_body,
        out_shape=(jax.ShapeDtypeStruct((N, 1), jnp.float32),
                   jax.ShapeDtypeStruct((N, 1), jnp.float32),
                   jax.ShapeDtypeStruct((N, D), jnp.float32)),
    )(degp, features)

    b1r, g1r, be1r = b1.reshape(1, H), g1.reshape(1, H), beta1.reshape(1, H)
    b2r, g2r, be2r = b2.reshape(1, H), g2.reshape(1, H), beta2.reshape(1, H)
    W3p = jnp.pad(W3, ((0, 0), (0, _W3COLS - O)))
    b3r = b3.reshape(1, O)

    p1 = _agg128(y1, src_g, dst_s, zagg)
    y2 = pl.pallas_call(
        _mid_body,
        out_shape=jax.ShapeDtypeStruct((N, D), jnp.float32),
    )(p1, nd, W1, b1r, g1r, be1r, ns)

    p2 = _agg128(y2, src_g, dst_s, zagg)
    z3 = pl.pallas_call(
        _mid3_body,
        out_shape=jax.ShapeDtypeStruct((N, _W3COLS), jnp.float32),
    )(p2, nd, W2, b2r, g2r, be2r, ns, W3p)

    p3 = _agg3(z3, src_g, dst_s, zagg16)
    out = pl.pallas_call(
        _fin_body,
        out_shape=jax.ShapeDtypeStruct((N, O), jnp.float32),
    )(p3, nd, b3r)
    return out


# speedup vs baseline: 10.6310x; 1.0671x over previous
import functools

import jax
import jax.numpy as jnp
from jax import lax
from jax.experimental import pallas as pl
from jax.experimental.pallas import tpu as pltpu
from jax.experimental.pallas import tpu_sc as plsc

N, E, D, H, O = 10000, 320000, 128, 128, 2
NC, NS = 2, 16
NW = NC * NS
WIN = 128
NPAD = 16
NROWS = 10112
NWIN = 2 * (-(-(-(-E // NW)) // (2 * WIN)))
EPW = NWIN * WIN
EP = EPW * NW
DEG_ROWS = 2 * NROWS
_DEG_RPS = DEG_ROWS // NS
_AGG_RPS = NROWS // NS

_mesh = plsc.VectorSubcoreMesh(core_axis_name="c", subcore_axis_name="s")


@functools.partial(
    pl.kernel,
    mesh=_mesh,
    out_type=jax.ShapeDtypeStruct((NC, DEG_ROWS, 16), jnp.float32),
    compiler_params=pltpu.CompilerParams(use_tc_tiling_on_sc=False),
    scratch_types=[
        pltpu.VMEM((NWIN, WIN), jnp.int32),
        pltpu.VMEM((NWIN, WIN), jnp.int32),
        pltpu.VMEM((WIN, 16), jnp.float32),
        pltpu.SemaphoreType.DMA,
        pltpu.VMEM_SHARED((DEG_ROWS, 16), jnp.float32),
    ],
)
def _deg_kernel(sidx_hbm, didx_hbm, ones_hbm, zeros_hbm, out_hbm,
                sidx_v, didx_v, ones_v, dsem, acc_sh):
    c = lax.axis_index("c")
    s = lax.axis_index("s")
    wid = c * NS + s
    r0 = s * _DEG_RPS
    pltpu.sync_copy(zeros_hbm, acc_sh.at[pl.ds(r0, _DEG_RPS)])
    pltpu.sync_copy(ones_hbm, ones_v)
    pltpu.sync_copy(sidx_hbm.at[wid], sidx_v)
    pltpu.sync_copy(didx_hbm.at[wid], didx_v)
    plsc.subcore_barrier()

    @pl.loop(0, NWIN)
    def _(j):
        pltpu.async_copy(ones_v, acc_sh.at[sidx_v.at[j]], dsem, add=True)
        pltpu.async_copy(ones_v, acc_sh.at[didx_v.at[j]], dsem, add=True)

        @pl.when(j > 0)
        def _():
            pltpu.make_async_copy(ones_v, acc_sh.at[sidx_v.at[j]],
                                  dsem).wait()
            pltpu.make_async_copy(ones_v, acc_sh.at[didx_v.at[j]],
                                  dsem).wait()

    pltpu.make_async_copy(ones_v, acc_sh.at[sidx_v.at[0]], dsem).wait()
    pltpu.make_async_copy(ones_v, acc_sh.at[didx_v.at[0]], dsem).wait()
    plsc.subcore_barrier()
    pltpu.sync_copy(acc_sh.at[pl.ds(r0, _DEG_RPS)],
                    out_hbm.at[c, pl.ds(r0, _DEG_RPS)])


def _make_agg_kernel(width):
    @functools.partial(
        pl.kernel,
        mesh=_mesh,
        out_type=jax.ShapeDtypeStruct((NC, NROWS, width), jnp.float32),
        compiler_params=pltpu.CompilerParams(use_tc_tiling_on_sc=False),
        scratch_types=[
            pltpu.VMEM((2, WIN), jnp.int32),
            pltpu.VMEM((NWIN, WIN), jnp.int32),
            pltpu.VMEM((WIN, width), jnp.float32),
            pltpu.VMEM((WIN, width), jnp.float32),
            pltpu.SemaphoreType.DMA,
            pltpu.SemaphoreType.DMA,
            pltpu.SemaphoreType.DMA,
            pltpu.SemaphoreType.DMA,
            pltpu.SemaphoreType.DMA,
            pltpu.SemaphoreType.DMA,
            pltpu.VMEM_SHARED((NROWS, width), jnp.float32),
        ],
    )
    def _agg_kernel(y_hbm, gidx_hbm, sidx_hbm, zeros_hbm, out_hbm,
                    gidx_v, sidx_v, rows0, rows1, semi0, semi1, semg0, semg1,
                    sems0, sems1, acc_sh):
        c = lax.axis_index("c")
        s = lax.axis_index("s")
        wid = c * NS + s
        r0 = s * _AGG_RPS
        pltpu.sync_copy(zeros_hbm, acc_sh.at[pl.ds(r0, _AGG_RPS)])
        pltpu.sync_copy(sidx_hbm.at[wid], sidx_v)

        def gidx_start(j, b, sem):
            pltpu.make_async_copy(gidx_hbm.at[wid, j], gidx_v.at[b],
                                  sem).start()

        def gidx_wait(j, b, sem):
            pltpu.make_async_copy(gidx_hbm.at[wid, j], gidx_v.at[b],
                                  sem).wait()

        plsc.subcore_barrier()

        gidx_start(0, 0, semi0)
        gidx_wait(0, 0, semi0)
        pltpu.make_async_copy(y_hbm.at[gidx_v.at[0]], rows0, semg0).start()
        gidx_start(1, 1, semi1)

        @pl.loop(0, NWIN, step=2)
        def _(j):
            @pl.when(j > 0)
            def _():
                pltpu.make_async_copy(rows1, acc_sh.at[sidx_v.at[j]],
                                      sems1).wait()

            pltpu.make_async_copy(y_hbm.at[gidx_v.at[0]], rows0,
                                  semg0).wait()
            gidx_wait(j + 1, 1, semi1)
            pltpu.make_async_copy(y_hbm.at[gidx_v.at[1]], rows1,
                                  semg1).start()
            pltpu.async_copy(rows0, acc_sh.at[sidx_v.at[j]], sems0, add=True)

            @pl.when(j + 2 < NWIN)
            def _():
                gidx_start(j + 2, 0, semi0)

            pltpu.make_async_copy(y_hbm.at[gidx_v.at[1]], rows1,
                                  semg1).wait()
            pltpu.make_async_copy(rows0, acc_sh.at[sidx_v.at[j]],
                                  sems0).wait()

            @pl.when(j + 2 < NWIN)
            def _():
                gidx_wait(j + 2, 0, semi0)
                pltpu.make_async_copy(y_hbm.at[gidx_v.at[0]], rows0,
                                      semg0).start()

            pltpu.async_copy(rows1, acc_sh.at[sidx_v.at[j + 1]], sems1,
                             add=True)

            @pl.when(j + 3 < NWIN)
            def _():
                gidx_start(j + 3, 1, semi1)

        pltpu.make_async_copy(rows1, acc_sh.at[sidx_v.at[NWIN - 1]],
                              sems1).wait()
        plsc.subcore_barrier()
        pltpu.sync_copy(acc_sh.at[pl.ds(r0, _AGG_RPS)],
                        out_hbm.at[c, pl.ds(r0, _AGG_RPS)])

    return _agg_kernel


_W3COLS = 16
_agg128 = _make_agg_kernel(D)
_agg3 = _agg128 if _W3COLS == D else _make_agg_kernel(_W3COLS)


def _prep_body(degp_ref, feat_ref, ns_ref, nd_ref, y1_ref):
    od = degp_ref[0, :N, 0:1] + degp_ref[1, :N, 0:1]
    idg = (degp_ref[0, NROWS:NROWS + N, 0:1]
           + degp_ref[1, NROWS:NROWS + N, 0:1])
    ns = jnp.where(od > 0, lax.rsqrt(od), 0.0)
    nd = jnp.where(idg > 0, lax.rsqrt(idg), 0.0)
    ns_ref[...] = ns
    nd_ref[...] = nd
    y1_ref[...] = feat_ref[...] * ns


def _f1_body(feat_ref, W_ref, o_ref):
    o_ref[...] = jnp.dot(feat_ref[...], W_ref[...],
                         preferred_element_type=jnp.float32)


def _mid_body(p_ref, nd_ref, b_ref, g_ref, be_ref, ns_ref, Wn_ref, o_ref):
    t = (p_ref[0, :N, :] + p_ref[1, :N, :]) * nd_ref[...] + b_ref[...]
    mu = jnp.mean(t, -1, keepdims=True)
    var = jnp.mean((t - mu) ** 2, -1, keepdims=True)
    x = (t - mu) / jnp.sqrt(var + 1e-5) * g_ref[...] + be_ref[...]
    x = jnp.where(x > 0, x, 0.01 * x)
    o_ref[...] = jnp.dot(x, Wn_ref[...],
                         preferred_element_type=jnp.float32) * ns_ref[...]


def _fin_body(p_ref, nd_ref, b3_ref, o_ref):
    a = p_ref[0, :N, :O] + p_ref[1, :N, :O]
    o_ref[...] = a * nd_ref[...] + b3_ref[...]


def kernel(features, edge_index, W1, b1, g1, beta1, W2, b2, g2, beta2, W3, b3):
    src = edge_index[0]
    dst = edge_index[1]

    pad = EP - E
    k = jnp.arange(pad, dtype=jnp.int32)
    kp = k % NPAD
    trash = N + kp
    src_g = jnp.concatenate([src, kp]).reshape(NW, NWIN, WIN)
    dst_s = jnp.concatenate([dst, trash]).reshape(NW, NWIN, WIN)
    src_d = jnp.concatenate([src, trash]).reshape(NW, NWIN, WIN)
    dst_d = jnp.concatenate([dst + NROWS, trash + NROWS]).reshape(NW, NWIN, WIN)

    ones16 = jnp.ones((WIN, 16), jnp.float32)
    zdeg = jnp.zeros((_DEG_RPS, 16), jnp.float32)
    zagg = jnp.zeros((_AGG_RPS, D), jnp.float32)
    zagg16 = jnp.zeros((_AGG_RPS, _W3COLS), jnp.float32)

    F1 = pl.pallas_call(
        _f1_body,
        out_shape=jax.ShapeDtypeStruct((N, H), jnp.float32),
    )(features, W1)
    degp = _deg_kernel(src_d, dst_d, ones16, zdeg)

    ns, nd, y1 = pl.pallas_call(
        _prep_body,
        out_shape=(jax.ShapeDtypeStruct((N, 1), jnp.float32),
                   jax.ShapeDtypeStruct((N, 1), jnp.float32),
                   jax.ShapeDtypeStruct((N, H), jnp.float32)),
    )(degp, F1)

    b1r, g1r, be1r = b1.reshape(1, H), g1.reshape(1, H), beta1.reshape(1, H)
    b2r, g2r, be2r = b2.reshape(1, H), g2.reshape(1, H), beta2.reshape(1, H)
    W3p = jnp.pad(W3, ((0, 0), (0, _W3COLS - O)))
    b3r = b3.reshape(1, O)

    p1 = _agg128(y1, src_g, dst_s, zagg)
    y2 = pl.pallas_call(
        _mid_body,
        out_shape=jax.ShapeDtypeStruct((N, H), jnp.float32),
    )(p1, nd, b1r, g1r, be1r, ns, W2)

    p2 = _agg128(y2, src_g, dst_s, zagg)
    z3 = pl.pallas_call(
        _mid_body,
        out_shape=jax.ShapeDtypeStruct((N, _W3COLS), jnp.float32),
    )(p2, nd, b2r, g2r, be2r, ns, W3p)

    p3 = _agg3(z3, src_g, dst_s, zagg16)
    out = pl.pallas_call(
        _fin_body,
        out_shape=jax.ShapeDtypeStruct((N, O), jnp.float32),
    )(p3, nd, b3r)
    return out


# speedup vs baseline: 11.2314x; 1.0565x over previous
import functools

import jax
import jax.numpy as jnp
from jax import lax
from jax.experimental import pallas as pl
from jax.experimental.pallas import tpu as pltpu
from jax.experimental.pallas import tpu_sc as plsc

N, E, D, H, O = 10000, 320000, 128, 128, 2
NC, NS = 2, 16
NW = NC * NS
WIN = 128
NPAD = 16
NROWS = 10112
NWIN = 2 * (-(-(-(-E // NW)) // (2 * WIN)))
EPW = NWIN * WIN
EP = EPW * NW
DEG_ROWS = 2 * NROWS
_DEG_RPS = DEG_ROWS // NS
_AGG_RPS = NROWS // NS

_mesh = plsc.VectorSubcoreMesh(core_axis_name="c", subcore_axis_name="s")


@functools.partial(
    pl.kernel,
    mesh=_mesh,
    out_type=jax.ShapeDtypeStruct((NC, DEG_ROWS, 16), jnp.float32),
    compiler_params=pltpu.CompilerParams(use_tc_tiling_on_sc=False),
    scratch_types=[
        pltpu.VMEM((NWIN, WIN), jnp.int32),
        pltpu.VMEM((NWIN, WIN), jnp.int32),
        pltpu.VMEM((WIN, 16), jnp.float32),
        pltpu.SemaphoreType.DMA,
        pltpu.VMEM_SHARED((DEG_ROWS, 16), jnp.float32),
    ],
)
def _deg_kernel(sidx_hbm, didx_hbm, ones_hbm, zeros_hbm, out_hbm,
                sidx_v, didx_v, ones_v, dsem, acc_sh):
    c = lax.axis_index("c")
    s = lax.axis_index("s")
    wid = c * NS + s
    r0 = s * _DEG_RPS
    pltpu.sync_copy(zeros_hbm, acc_sh.at[pl.ds(r0, _DEG_RPS)])
    pltpu.sync_copy(ones_hbm, ones_v)
    pltpu.sync_copy(sidx_hbm.at[wid], sidx_v)
    pltpu.sync_copy(didx_hbm.at[wid], didx_v)
    plsc.subcore_barrier()

    @pl.loop(0, NWIN)
    def _(j):
        pltpu.async_copy(ones_v, acc_sh.at[sidx_v.at[j]], dsem, add=True)
        pltpu.async_copy(ones_v, acc_sh.at[didx_v.at[j]], dsem, add=True)

        @pl.when(j > 0)
        def _():
            pltpu.make_async_copy(ones_v, acc_sh.at[sidx_v.at[j]],
                                  dsem).wait()
            pltpu.make_async_copy(ones_v, acc_sh.at[didx_v.at[j]],
                                  dsem).wait()

    pltpu.make_async_copy(ones_v, acc_sh.at[sidx_v.at[0]], dsem).wait()
    pltpu.make_async_copy(ones_v, acc_sh.at[didx_v.at[0]], dsem).wait()
    plsc.subcore_barrier()
    pltpu.sync_copy(acc_sh.at[pl.ds(r0, _DEG_RPS)],
                    out_hbm.at[c, pl.ds(r0, _DEG_RPS)])


def _make_agg_kernel(width):
    @functools.partial(
        pl.kernel,
        mesh=_mesh,
        out_type=jax.ShapeDtypeStruct((NC, NROWS, width), jnp.float32),
        compiler_params=pltpu.CompilerParams(use_tc_tiling_on_sc=False),
        scratch_types=[
            pltpu.VMEM((2, WIN), jnp.int32),
            pltpu.VMEM((NWIN, WIN), jnp.int32),
            pltpu.VMEM((WIN, width), jnp.float32),
            pltpu.VMEM((WIN, width), jnp.float32),
            pltpu.SemaphoreType.DMA,
            pltpu.SemaphoreType.DMA,
            pltpu.SemaphoreType.DMA,
            pltpu.SemaphoreType.DMA,
            pltpu.SemaphoreType.DMA,
            pltpu.SemaphoreType.DMA,
            pltpu.VMEM_SHARED((NROWS, width), jnp.float32),
        ],
    )
    def _agg_kernel(y_hbm, gidx_hbm, sidx_hbm, zeros_hbm, out_hbm,
                    gidx_v, sidx_v, rows0, rows1, semi0, semi1, semg0, semg1,
                    sems0, sems1, acc_sh):
        c = lax.axis_index("c")
        s = lax.axis_index("s")
        wid = c * NS + s
        r0 = s * _AGG_RPS
        pltpu.sync_copy(zeros_hbm, acc_sh.at[pl.ds(r0, _AGG_RPS)])
        pltpu.sync_copy(sidx_hbm.at[wid], sidx_v)

        def gidx_start(j, b, sem):
            pltpu.make_async_copy(gidx_hbm.at[wid, j], gidx_v.at[b],
                                  sem).start()

        def gidx_wait(j, b, sem):
            pltpu.make_async_copy(gidx_hbm.at[wid, j], gidx_v.at[b],
                                  sem).wait()

        plsc.subcore_barrier()

        gidx_start(0, 0, semi0)
        gidx_wait(0, 0, semi0)
        pltpu.make_async_copy(y_hbm.at[gidx_v.at[0]], rows0, semg0).start()
        gidx_start(1, 1, semi1)

        @pl.loop(0, NWIN, step=2)
        def _(j):
            @pl.when(j > 0)
            def _():
                pltpu.make_async_copy(rows1, acc_sh.at[sidx_v.at[j]],
                                      sems1).wait()

            pltpu.make_async_copy(y_hbm.at[gidx_v.at[0]], rows0,
                                  semg0).wait()
            gidx_wait(j + 1, 1, semi1)
            pltpu.make_async_copy(y_hbm.at[gidx_v.at[1]], rows1,
                                  semg1).start()
            pltpu.async_copy(rows0, acc_sh.at[sidx_v.at[j]], sems0, add=True)

            @pl.when(j + 2 < NWIN)
            def _():
                gidx_start(j + 2, 0, semi0)

            pltpu.make_async_copy(y_hbm.at[gidx_v.at[1]], rows1,
                                  semg1).wait()
            pltpu.make_async_copy(rows0, acc_sh.at[sidx_v.at[j]],
                                  sems0).wait()

            @pl.when(j + 2 < NWIN)
            def _():
                gidx_wait(j + 2, 0, semi0)
                pltpu.make_async_copy(y_hbm.at[gidx_v.at[0]], rows0,
                                      semg0).start()

            pltpu.async_copy(rows1, acc_sh.at[sidx_v.at[j + 1]], sems1,
                             add=True)

            @pl.when(j + 3 < NWIN)
            def _():
                gidx_start(j + 3, 1, semi1)

        pltpu.make_async_copy(rows1, acc_sh.at[sidx_v.at[NWIN - 1]],
                              sems1).wait()
        plsc.subcore_barrier()
        pltpu.sync_copy(acc_sh.at[pl.ds(r0, _AGG_RPS)],
                        out_hbm.at[c, pl.ds(r0, _AGG_RPS)])

    return _agg_kernel


def _make_agg_kernel_deep(width):
    @functools.partial(
        pl.kernel,
        mesh=_mesh,
        out_type=jax.ShapeDtypeStruct((NC, NROWS, width), jnp.float32),
        compiler_params=pltpu.CompilerParams(use_tc_tiling_on_sc=False),
        scratch_types=(
            [pltpu.VMEM((4, WIN), jnp.int32),
             pltpu.VMEM((NWIN, WIN), jnp.int32)]
            + [pltpu.VMEM((WIN, width), jnp.float32) for _ in range(4)]
            + [pltpu.SemaphoreType.DMA for _ in range(12)]
            + [pltpu.VMEM_SHARED((NROWS, width), jnp.float32)]
        ),
    )
    def _agg_kernel(y_hbm, gidx_hbm, sidx_hbm, zeros_hbm, out_hbm,
                    gidx_v, sidx_v, r0b, r1b, r2b, r3b,
                    i0, i1, i2, i3, g0, g1, g2, g3, s0, s1, s2, s3,
                    acc_sh):
        rows = [r0b, r1b, r2b, r3b]
        semi = [i0, i1, i2, i3]
        semg = [g0, g1, g2, g3]
        sems = [s0, s1, s2, s3]
        c = lax.axis_index("c")
        s = lax.axis_index("s")
        wid = c * NS + s
        rb = s * _AGG_RPS
        pltpu.sync_copy(zeros_hbm, acc_sh.at[pl.ds(rb, _AGG_RPS)])
        pltpu.sync_copy(sidx_hbm.at[wid], sidx_v)

        def gi_start(w, b):
            pltpu.make_async_copy(gidx_hbm.at[wid, w], gidx_v.at[b],
                                  semi[b]).start()

        def gi_wait(w, b):
            pltpu.make_async_copy(gidx_hbm.at[wid, w], gidx_v.at[b],
                                  semi[b]).wait()

        def g_start(b):
            pltpu.make_async_copy(y_hbm.at[gidx_v.at[b]], rows[b],
                                  semg[b]).start()

        def g_wait(b):
            pltpu.make_async_copy(y_hbm.at[gidx_v.at[b]], rows[b],
                                  semg[b]).wait()

        def s_start(w, b):
            pltpu.async_copy(rows[b], acc_sh.at[sidx_v.at[w]], sems[b],
                             add=True)

        def s_wait(w, b):
            pltpu.make_async_copy(rows[b], acc_sh.at[sidx_v.at[w]],
                                  sems[b]).wait()

        plsc.subcore_barrier()

        for b in range(4):
            gi_start(b, b)
        for b in range(2):
            gi_wait(b, b)
            g_start(b)

        @pl.loop(0, NWIN, step=4)
        def _(j):
            for b in range(4):
                w = j + b

                @pl.when(w >= 2)
                def _():
                    s_wait(w - 2, (b + 2) % 4)

                g_wait(b)

                @pl.when(w + 4 < NWIN)
                def _():
                    gi_start(w + 4, b)

                @pl.when(w + 2 < NWIN)
                def _():
                    gi_wait(w + 2, (b + 2) % 4)
                    g_start((b + 2) % 4)

                s_start(w, b)

        s_wait(NWIN - 2, (NWIN - 2) % 4)
        s_wait(NWIN - 1, (NWIN - 1) % 4)
        plsc.subcore_barrier()
        pltpu.sync_copy(acc_sh.at[pl.ds(rb, _AGG_RPS)],
                        out_hbm.at[c, pl.ds(rb, _AGG_RPS)])

    return _agg_kernel


_W3COLS = 16
_agg128 = _make_agg_kernel(D)
_agg3 = _agg128 if _W3COLS == D else _make_agg_kernel_deep(_W3COLS)


def _prep_body(degp_ref, feat_ref, ns_ref, nd_ref, y1_ref):
    od = degp_ref[0, :N, 0:1] + degp_ref[1, :N, 0:1]
    idg = (degp_ref[0, NROWS:NROWS + N, 0:1]
           + degp_ref[1, NROWS:NROWS + N, 0:1])
    ns = jnp.where(od > 0, lax.rsqrt(od), 0.0)
    nd = jnp.where(idg > 0, lax.rsqrt(idg), 0.0)
    ns_ref[...] = ns
    nd_ref[...] = nd
    y1_ref[...] = feat_ref[...] * ns


def _f1_body(feat_ref, W_ref, o_ref):
    o_ref[...] = jnp.dot(feat_ref[...], W_ref[...],
                         preferred_element_type=jnp.float32)


def _mid_body(p_ref, nd_ref, b_ref, g_ref, be_ref, ns_ref, Wn_ref, o_ref):
    t = (p_ref[0, :N, :] + p_ref[1, :N, :]) * nd_ref[...] + b_ref[...]
    mu = jnp.mean(t, -1, keepdims=True)
    var = jnp.mean((t - mu) ** 2, -1, keepdims=True)
    x = (t - mu) / jnp.sqrt(var + 1e-5) * g_ref[...] + be_ref[...]
    x = jnp.where(x > 0, x, 0.01 * x)
    o_ref[...] = jnp.dot(x, Wn_ref[...],
                         preferred_element_type=jnp.float32) * ns_ref[...]


def _fin_body(p_ref, nd_ref, b3_ref, o_ref):
    a = p_ref[0, :N, :O] + p_ref[1, :N, :O]
    o_ref[...] = a * nd_ref[...] + b3_ref[...]


def kernel(features, edge_index, W1, b1, g1, beta1, W2, b2, g2, beta2, W3, b3):
    src = edge_index[0]
    dst = edge_index[1]

    pad = EP - E
    k = jnp.arange(pad, dtype=jnp.int32)
    kp = k % NPAD
    trash = N + kp
    src_g = jnp.concatenate([src, kp]).reshape(NW, NWIN, WIN)
    dst_s = jnp.concatenate([dst, trash]).reshape(NW, NWIN, WIN)
    src_d = jnp.concatenate([src, trash]).reshape(NW, NWIN, WIN)
    dst_d = jnp.concatenate([dst + NROWS, trash + NROWS]).reshape(NW, NWIN, WIN)

    ones16 = jnp.ones((WIN, 16), jnp.float32)
    zdeg = jnp.zeros((_DEG_RPS, 16), jnp.float32)
    zagg = jnp.zeros((_AGG_RPS, D), jnp.float32)
    zagg16 = jnp.zeros((_AGG_RPS, _W3COLS), jnp.float32)

    F1 = pl.pallas_call(
        _f1_body,
        out_shape=jax.ShapeDtypeStruct((N, H), jnp.float32),
    )(features, W1)
    degp = _deg_kernel(src_d, dst_d, ones16, zdeg)

    ns, nd, y1 = pl.pallas_call(
        _prep_body,
        out_shape=(jax.ShapeDtypeStruct((N, 1), jnp.float32),
                   jax.ShapeDtypeStruct((N, 1), jnp.float32),
                   jax.ShapeDtypeStruct((N, H), jnp.float32)),
    )(degp, F1)

    b1r, g1r, be1r = b1.reshape(1, H), g1.reshape(1, H), beta1.reshape(1, H)
    b2r, g2r, be2r = b2.reshape(1, H), g2.reshape(1, H), beta2.reshape(1, H)
    W3p = jnp.pad(W3, ((0, 0), (0, _W3COLS - O)))
    b3r = b3.reshape(1, O)

    p1 = _agg128(y1, src_g, dst_s, zagg)
    y2 = pl.pallas_call(
        _mid_body,
        out_shape=jax.ShapeDtypeStruct((N, H), jnp.float32),
    )(p1, nd, b1r, g1r, be1r, ns, W2)

    p2 = _agg128(y2, src_g, dst_s, zagg)
    z3 = pl.pallas_call(
        _mid_body,
        out_shape=jax.ShapeDtypeStruct((N, _W3COLS), jnp.float32),
    )(p2, nd, b2r, g2r, be2r, ns, W3p)

    p3 = _agg3(z3, src_g, dst_s, zagg16)
    out = pl.pallas_call(
        _fin_body,
        out_shape=jax.ShapeDtypeStruct((N, O), jnp.float32),
    )(p3, nd, b3r)
    return out


# speedup vs baseline: 11.3729x; 1.0126x over previous
import functools

import jax
import jax.numpy as jnp
from jax import lax
from jax.experimental import pallas as pl
from jax.experimental.pallas import tpu as pltpu
from jax.experimental.pallas import tpu_sc as plsc

N, E, D, H, O = 10000, 320000, 128, 128, 2
NC, NS = 2, 16
NW = NC * NS
WIN = 128
NPAD = 16
NROWS = 10112
NWIN = 2 * (-(-(-(-E // NW)) // (2 * WIN)))
EPW = NWIN * WIN
EP = EPW * NW
DEG_ROWS = 2 * NROWS
_DEG_RPS = DEG_ROWS // NS
_AGG_RPS = NROWS // NS

_mesh = plsc.VectorSubcoreMesh(core_axis_name="c", subcore_axis_name="s")


@functools.partial(
    pl.kernel,
    mesh=_mesh,
    out_type=jax.ShapeDtypeStruct((NC, DEG_ROWS, 16), jnp.float32),
    compiler_params=pltpu.CompilerParams(use_tc_tiling_on_sc=False),
    scratch_types=[
        pltpu.VMEM((NWIN, WIN), jnp.int32),
        pltpu.VMEM((NWIN, WIN), jnp.int32),
        pltpu.VMEM((WIN, 16), jnp.float32),
        pltpu.SemaphoreType.DMA,
        pltpu.VMEM_SHARED((DEG_ROWS, 16), jnp.float32),
    ],
)
def _deg_kernel(sidx_hbm, didx_hbm, ones_hbm, zeros_hbm, out_hbm,
                sidx_v, didx_v, ones_v, dsem, acc_sh):
    c = lax.axis_index("c")
    s = lax.axis_index("s")
    wid = c * NS + s
    r0 = s * _DEG_RPS
    pltpu.sync_copy(zeros_hbm, acc_sh.at[pl.ds(r0, _DEG_RPS)])
    pltpu.sync_copy(ones_hbm, ones_v)
    pltpu.sync_copy(sidx_hbm.at[wid], sidx_v)
    pltpu.sync_copy(didx_hbm.at[wid], didx_v)
    plsc.subcore_barrier()

    @pl.loop(0, NWIN)
    def _(j):
        pltpu.async_copy(ones_v, acc_sh.at[sidx_v.at[j]], dsem, add=True)
        pltpu.async_copy(ones_v, acc_sh.at[didx_v.at[j]], dsem, add=True)

        @pl.when(j > 0)
        def _():
            pltpu.make_async_copy(ones_v, acc_sh.at[sidx_v.at[j]],
                                  dsem).wait()
            pltpu.make_async_copy(ones_v, acc_sh.at[didx_v.at[j]],
                                  dsem).wait()

    pltpu.make_async_copy(ones_v, acc_sh.at[sidx_v.at[0]], dsem).wait()
    pltpu.make_async_copy(ones_v, acc_sh.at[didx_v.at[0]], dsem).wait()
    plsc.subcore_barrier()
    pltpu.sync_copy(acc_sh.at[pl.ds(r0, _DEG_RPS)],
                    out_hbm.at[c, pl.ds(r0, _DEG_RPS)])


def _make_agg_kernel(width, dtype=jnp.float32):
    @functools.partial(
        pl.kernel,
        mesh=_mesh,
        out_type=jax.ShapeDtypeStruct((NC, NROWS, width), dtype),
        compiler_params=pltpu.CompilerParams(use_tc_tiling_on_sc=False),
        scratch_types=[
            pltpu.VMEM((2, WIN), jnp.int32),
            pltpu.VMEM((NWIN, WIN), jnp.int32),
            pltpu.VMEM((WIN, width), dtype),
            pltpu.VMEM((WIN, width), dtype),
            pltpu.SemaphoreType.DMA,
            pltpu.SemaphoreType.DMA,
            pltpu.SemaphoreType.DMA,
            pltpu.SemaphoreType.DMA,
            pltpu.SemaphoreType.DMA,
            pltpu.SemaphoreType.DMA,
            pltpu.VMEM_SHARED((NROWS, width), dtype),
        ],
    )
    def _agg_kernel(y_hbm, gidx_hbm, sidx_hbm, zeros_hbm, out_hbm,
                    gidx_v, sidx_v, rows0, rows1, semi0, semi1, semg0, semg1,
                    sems0, sems1, acc_sh):
        c = lax.axis_index("c")
        s = lax.axis_index("s")
        wid = c * NS + s
        r0 = s * _AGG_RPS
        pltpu.sync_copy(zeros_hbm, acc_sh.at[pl.ds(r0, _AGG_RPS)])
        pltpu.sync_copy(sidx_hbm.at[wid], sidx_v)

        def gidx_start(j, b, sem):
            pltpu.make_async_copy(gidx_hbm.at[wid, j], gidx_v.at[b],
                                  sem).start()

        def gidx_wait(j, b, sem):
            pltpu.make_async_copy(gidx_hbm.at[wid, j], gidx_v.at[b],
                                  sem).wait()

        plsc.subcore_barrier()

        gidx_start(0, 0, semi0)
        gidx_wait(0, 0, semi0)
        pltpu.make_async_copy(y_hbm.at[gidx_v.at[0]], rows0, semg0).start()
        gidx_start(1, 1, semi1)

        @pl.loop(0, NWIN, step=2)
        def _(j):
            @pl.when(j > 0)
            def _():
                pltpu.make_async_copy(rows1, acc_sh.at[sidx_v.at[j]],
                                      sems1).wait()

            pltpu.make_async_copy(y_hbm.at[gidx_v.at[0]], rows0,
                                  semg0).wait()
            gidx_wait(j + 1, 1, semi1)
            pltpu.make_async_copy(y_hbm.at[gidx_v.at[1]], rows1,
                                  semg1).start()
            pltpu.async_copy(rows0, acc_sh.at[sidx_v.at[j]], sems0, add=True)

            @pl.when(j + 2 < NWIN)
            def _():
                gidx_start(j + 2, 0, semi0)

            pltpu.make_async_copy(y_hbm.at[gidx_v.at[1]], rows1,
                                  semg1).wait()
            pltpu.make_async_copy(rows0, acc_sh.at[sidx_v.at[j]],
                                  sems0).wait()

            @pl.when(j + 2 < NWIN)
            def _():
                gidx_wait(j + 2, 0, semi0)
                pltpu.make_async_copy(y_hbm.at[gidx_v.at[0]], rows0,
                                      semg0).start()

            pltpu.async_copy(rows1, acc_sh.at[sidx_v.at[j + 1]], sems1,
                             add=True)

            @pl.when(j + 3 < NWIN)
            def _():
                gidx_start(j + 3, 1, semi1)

        pltpu.make_async_copy(rows1, acc_sh.at[sidx_v.at[NWIN - 1]],
                              sems1).wait()
        plsc.subcore_barrier()
        pltpu.sync_copy(acc_sh.at[pl.ds(r0, _AGG_RPS)],
                        out_hbm.at[c, pl.ds(r0, _AGG_RPS)])

    return _agg_kernel


def _make_agg_kernel_deep(width):
    @functools.partial(
        pl.kernel,
        mesh=_mesh,
        out_type=jax.ShapeDtypeStruct((NC, NROWS, width), jnp.float32),
        compiler_params=pltpu.CompilerParams(use_tc_tiling_on_sc=False),
        scratch_types=(
            [pltpu.VMEM((4, WIN), jnp.int32),
             pltpu.VMEM((NWIN, WIN), jnp.int32)]
            + [pltpu.VMEM((WIN, width), jnp.float32) for _ in range(4)]
            + [pltpu.SemaphoreType.DMA for _ in range(12)]
            + [pltpu.VMEM_SHARED((NROWS, width), jnp.float32)]
        ),
    )
    def _agg_kernel(y_hbm, gidx_hbm, sidx_hbm, zeros_hbm, out_hbm,
                    gidx_v, sidx_v, r0b, r1b, r2b, r3b,
                    i0, i1, i2, i3, g0, g1, g2, g3, s0, s1, s2, s3,
                    acc_sh):
        rows = [r0b, r1b, r2b, r3b]
        semi = [i0, i1, i2, i3]
        semg = [g0, g1, g2, g3]
        sems = [s0, s1, s2, s3]
        c = lax.axis_index("c")
        s = lax.axis_index("s")
        wid = c * NS + s
        rb = s * _AGG_RPS
        pltpu.sync_copy(zeros_hbm, acc_sh.at[pl.ds(rb, _AGG_RPS)])
        pltpu.sync_copy(sidx_hbm.at[wid], sidx_v)

        def gi_start(w, b):
            pltpu.make_async_copy(gidx_hbm.at[wid, w], gidx_v.at[b],
                                  semi[b]).start()

        def gi_wait(w, b):
            pltpu.make_async_copy(gidx_hbm.at[wid, w], gidx_v.at[b],
                                  semi[b]).wait()

        def g_start(b):
            pltpu.make_async_copy(y_hbm.at[gidx_v.at[b]], rows[b],
                                  semg[b]).start()

        def g_wait(b):
            pltpu.make_async_copy(y_hbm.at[gidx_v.at[b]], rows[b],
                                  semg[b]).wait()

        def s_start(w, b):
            pltpu.async_copy(rows[b], acc_sh.at[sidx_v.at[w]], sems[b],
                             add=True)

        def s_wait(w, b):
            pltpu.make_async_copy(rows[b], acc_sh.at[sidx_v.at[w]],
                                  sems[b]).wait()

        plsc.subcore_barrier()

        for b in range(4):
            gi_start(b, b)
        for b in range(2):
            gi_wait(b, b)
            g_start(b)

        @pl.loop(0, NWIN, step=4)
        def _(j):
            for b in range(4):
                w = j + b

                @pl.when(w >= 2)
                def _():
                    s_wait(w - 2, (b + 2) % 4)

                g_wait(b)

                @pl.when(w + 4 < NWIN)
                def _():
                    gi_start(w + 4, b)

                @pl.when(w + 2 < NWIN)
                def _():
                    gi_wait(w + 2, (b + 2) % 4)
                    g_start((b + 2) % 4)

                s_start(w, b)

        s_wait(NWIN - 2, (NWIN - 2) % 4)
        s_wait(NWIN - 1, (NWIN - 1) % 4)
        plsc.subcore_barrier()
        pltpu.sync_copy(acc_sh.at[pl.ds(rb, _AGG_RPS)],
                        out_hbm.at[c, pl.ds(rb, _AGG_RPS)])

    return _agg_kernel


_W3COLS = 16
_agg128 = _make_agg_kernel(D, jnp.bfloat16)
_agg3 = _agg128 if _W3COLS == D else _make_agg_kernel_deep(_W3COLS)


def _prep_body(degp_ref, feat_ref, ns_ref, nd_ref, y1_ref):
    od = degp_ref[0, :N, 0:1] + degp_ref[1, :N, 0:1]
    idg = (degp_ref[0, NROWS:NROWS + N, 0:1]
           + degp_ref[1, NROWS:NROWS + N, 0:1])
    ns = jnp.where(od > 0, lax.rsqrt(od), 0.0)
    nd = jnp.where(idg > 0, lax.rsqrt(idg), 0.0)
    ns_ref[...] = ns
    nd_ref[...] = nd
    y1_ref[...] = (feat_ref[...] * ns).astype(y1_ref.dtype)


def _f1_body(feat_ref, W_ref, o_ref):
    o_ref[...] = jnp.dot(feat_ref[...], W_ref[...],
                         preferred_element_type=jnp.float32)


def _mid_body(p_ref, nd_ref, b_ref, g_ref, be_ref, ns_ref, Wn_ref, o_ref):
    agg = (p_ref[0, :N, :].astype(jnp.float32)
           + p_ref[1, :N, :].astype(jnp.float32))
    t = agg * nd_ref[...] + b_ref[...]
    mu = jnp.mean(t, -1, keepdims=True)
    var = jnp.mean((t - mu) ** 2, -1, keepdims=True)
    x = (t - mu) / jnp.sqrt(var + 1e-5) * g_ref[...] + be_ref[...]
    x = jnp.where(x > 0, x, 0.01 * x)
    y = jnp.dot(x, Wn_ref[...],
                preferred_element_type=jnp.float32) * ns_ref[...]
    o_ref[...] = y.astype(o_ref.dtype)


def _fin_body(p_ref, nd_ref, b3_ref, o_ref):
    a = p_ref[0, :N, :O] + p_ref[1, :N, :O]
    o_ref[...] = a * nd_ref[...] + b3_ref[...]


def kernel(features, edge_index, W1, b1, g1, beta1, W2, b2, g2, beta2, W3, b3):
    src = edge_index[0]
    dst = edge_index[1]

    pad = EP - E
    k = jnp.arange(pad, dtype=jnp.int32)
    kp = k % NPAD
    trash = N + kp
    src_g = jnp.concatenate([src, kp]).reshape(NW, NWIN, WIN)
    dst_s = jnp.concatenate([dst, trash]).reshape(NW, NWIN, WIN)
    src_d = jnp.concatenate([src, trash]).reshape(NW, NWIN, WIN)
    dst_d = jnp.concatenate([dst + NROWS, trash + NROWS]).reshape(NW, NWIN, WIN)

    ones16 = jnp.ones((WIN, 16), jnp.float32)
    zdeg = jnp.zeros((_DEG_RPS, 16), jnp.float32)
    zagg = jnp.zeros((_AGG_RPS, D), jnp.bfloat16)
    zagg16 = jnp.zeros((_AGG_RPS, _W3COLS), jnp.float32)

    F1 = pl.pallas_call(
        _f1_body,
        out_shape=jax.ShapeDtypeStruct((N, H), jnp.float32),
    )(features, W1)
    degp = _deg_kernel(src_d, dst_d, ones16, zdeg)

    ns, nd, y1 = pl.pallas_call(
        _prep_body,
        out_shape=(jax.ShapeDtypeStruct((N, 1), jnp.float32),
                   jax.ShapeDtypeStruct((N, 1), jnp.float32),
                   jax.ShapeDtypeStruct((N, H), jnp.bfloat16)),
    )(degp, F1)

    b1r, g1r, be1r = b1.reshape(1, H), g1.reshape(1, H), beta1.reshape(1, H)
    b2r, g2r, be2r = b2.reshape(1, H), g2.reshape(1, H), beta2.reshape(1, H)
    W3p = jnp.pad(W3, ((0, 0), (0, _W3COLS - O)))
    b3r = b3.reshape(1, O)

    p1 = _agg128(y1, src_g, dst_s, zagg)
    y2 = pl.pallas_call(
        _mid_body,
        out_shape=jax.ShapeDtypeStruct((N, H), jnp.bfloat16),
    )(p1, nd, b1r, g1r, be1r, ns, W2)

    p2 = _agg128(y2, src_g, dst_s, zagg)
    z3 = pl.pallas_call(
        _mid_body,
        out_shape=jax.ShapeDtypeStruct((N, _W3COLS), jnp.float32),
    )(p2, nd, b2r, g2r, be2r, ns, W3p)

    p3 = _agg3(z3, src_g, dst_s, zagg16)
    out = pl.pallas_call(
        _fin_body,
        out_shape=jax.ShapeDtypeStruct((N, O), jnp.float32),
    )(p3, nd, b3r)
    return out


# speedup vs baseline: 11.3964x; 1.0021x over previous
import functools

import jax
import jax.numpy as jnp
from jax import lax
from jax.experimental import pallas as pl
from jax.experimental.pallas import tpu as pltpu
from jax.experimental.pallas import tpu_sc as plsc

N, E, D, H, O = 10000, 320000, 128, 128, 2
NC, NS = 2, 16
NW = NC * NS
WIN = 128
NPAD = 16
NROWS = 10112
NWIN = 2 * (-(-(-(-E // NW)) // (2 * WIN)))
EPW = NWIN * WIN
EP = EPW * NW
DEG_ROWS = 2 * NROWS
_DEG_RPS = DEG_ROWS // NS
_AGG_RPS = NROWS // NS

_mesh = plsc.VectorSubcoreMesh(core_axis_name="c", subcore_axis_name="s")


@functools.partial(
    pl.kernel,
    mesh=_mesh,
    out_type=jax.ShapeDtypeStruct((NC, DEG_ROWS, 16), jnp.float32),
    compiler_params=pltpu.CompilerParams(use_tc_tiling_on_sc=False),
    scratch_types=[
        pltpu.VMEM((NWIN, WIN), jnp.int32),
        pltpu.VMEM((NWIN, WIN), jnp.int32),
        pltpu.VMEM((WIN, 16), jnp.float32),
        pltpu.SemaphoreType.DMA,
        pltpu.VMEM_SHARED((DEG_ROWS, 16), jnp.float32),
    ],
)
def _deg_kernel(sidx_hbm, didx_hbm, ones_hbm, zeros_hbm, out_hbm,
                sidx_v, didx_v, ones_v, dsem, acc_sh):
    c = lax.axis_index("c")
    s = lax.axis_index("s")
    wid = c * NS + s
    r0 = s * _DEG_RPS
    pltpu.sync_copy(zeros_hbm, acc_sh.at[pl.ds(r0, _DEG_RPS)])
    pltpu.sync_copy(ones_hbm, ones_v)
    pltpu.sync_copy(sidx_hbm.at[wid], sidx_v)
    pltpu.sync_copy(didx_hbm.at[wid], didx_v)
    plsc.subcore_barrier()

    @pl.loop(0, NWIN)
    def _(j):
        pltpu.async_copy(ones_v, acc_sh.at[sidx_v.at[j]], dsem, add=True)
        pltpu.async_copy(ones_v, acc_sh.at[didx_v.at[j]], dsem, add=True)

        @pl.when(j > 0)
        def _():
            pltpu.make_async_copy(ones_v, acc_sh.at[sidx_v.at[j]],
                                  dsem).wait()
            pltpu.make_async_copy(ones_v, acc_sh.at[didx_v.at[j]],
                                  dsem).wait()

    pltpu.make_async_copy(ones_v, acc_sh.at[sidx_v.at[0]], dsem).wait()
    pltpu.make_async_copy(ones_v, acc_sh.at[didx_v.at[0]], dsem).wait()
    plsc.subcore_barrier()
    pltpu.sync_copy(acc_sh.at[pl.ds(r0, _DEG_RPS)],
                    out_hbm.at[c, pl.ds(r0, _DEG_RPS)])


def _make_agg_kernel(width, dtype=jnp.float32):
    @functools.partial(
        pl.kernel,
        mesh=_mesh,
        out_type=jax.ShapeDtypeStruct((NC, NROWS, width), dtype),
        compiler_params=pltpu.CompilerParams(use_tc_tiling_on_sc=False),
        scratch_types=[
            pltpu.VMEM((2, WIN), jnp.int32),
            pltpu.VMEM((NWIN, WIN), jnp.int32),
            pltpu.VMEM((WIN, width), dtype),
            pltpu.VMEM((WIN, width), dtype),
            pltpu.SemaphoreType.DMA,
            pltpu.SemaphoreType.DMA,
            pltpu.SemaphoreType.DMA,
            pltpu.SemaphoreType.DMA,
            pltpu.SemaphoreType.DMA,
            pltpu.SemaphoreType.DMA,
            pltpu.VMEM_SHARED((NROWS, width), dtype),
        ],
    )
    def _agg_kernel(y_hbm, gidx_hbm, sidx_hbm, zeros_hbm, out_hbm,
                    gidx_v, sidx_v, rows0, rows1, semi0, semi1, semg0, semg1,
                    sems0, sems1, acc_sh):
        c = lax.axis_index("c")
        s = lax.axis_index("s")
        wid = c * NS + s
        r0 = s * _AGG_RPS
        pltpu.sync_copy(zeros_hbm, acc_sh.at[pl.ds(r0, _AGG_RPS)])
        pltpu.sync_copy(sidx_hbm.at[wid], sidx_v)

        def gidx_start(j, b, sem):
            pltpu.make_async_copy(gidx_hbm.at[wid, j], gidx_v.at[b],
                                  sem).start()

        def gidx_wait(j, b, sem):
            pltpu.make_async_copy(gidx_hbm.at[wid, j], gidx_v.at[b],
                                  sem).wait()

        plsc.subcore_barrier()

        gidx_start(0, 0, semi0)
        gidx_wait(0, 0, semi0)
        pltpu.make_async_copy(y_hbm.at[gidx_v.at[0]], rows0, semg0).start()
        gidx_start(1, 1, semi1)

        @pl.loop(0, NWIN, step=2)
        def _(j):
            @pl.when(j > 0)
            def _():
                pltpu.make_async_copy(rows1, acc_sh.at[sidx_v.at[j]],
                                      sems1).wait()

            pltpu.make_async_copy(y_hbm.at[gidx_v.at[0]], rows0,
                                  semg0).wait()
            gidx_wait(j + 1, 1, semi1)
            pltpu.make_async_copy(y_hbm.at[gidx_v.at[1]], rows1,
                                  semg1).start()
            pltpu.async_copy(rows0, acc_sh.at[sidx_v.at[j]], sems0, add=True)

            @pl.when(j + 2 < NWIN)
            def _():
                gidx_start(j + 2, 0, semi0)

            pltpu.make_async_copy(y_hbm.at[gidx_v.at[1]], rows1,
                                  semg1).wait()
            pltpu.make_async_copy(rows0, acc_sh.at[sidx_v.at[j]],
                                  sems0).wait()

            @pl.when(j + 2 < NWIN)
            def _():
                gidx_wait(j + 2, 0, semi0)
                pltpu.make_async_copy(y_hbm.at[gidx_v.at[0]], rows0,
                                      semg0).start()

            pltpu.async_copy(rows1, acc_sh.at[sidx_v.at[j + 1]], sems1,
                             add=True)

            @pl.when(j + 3 < NWIN)
            def _():
                gidx_start(j + 3, 1, semi1)

        pltpu.make_async_copy(rows1, acc_sh.at[sidx_v.at[NWIN - 1]],
                              sems1).wait()
        plsc.subcore_barrier()
        pltpu.sync_copy(acc_sh.at[pl.ds(r0, _AGG_RPS)],
                        out_hbm.at[c, pl.ds(r0, _AGG_RPS)])

    return _agg_kernel


def _make_agg_kernel_deep(width):
    @functools.partial(
        pl.kernel,
        mesh=_mesh,
        out_type=jax.ShapeDtypeStruct((NC, NROWS, width), jnp.float32),
        compiler_params=pltpu.CompilerParams(use_tc_tiling_on_sc=False),
        scratch_types=(
            [pltpu.VMEM((4, WIN), jnp.int32),
             pltpu.VMEM((NWIN, WIN), jnp.int32)]
            + [pltpu.VMEM((WIN, width), jnp.float32) for _ in range(4)]
            + [pltpu.SemaphoreType.DMA for _ in range(12)]
            + [pltpu.VMEM_SHARED((NROWS, width), jnp.float32)]
        ),
    )
    def _agg_kernel(y_hbm, gidx_hbm, sidx_hbm, zeros_hbm, out_hbm,
                    gidx_v, sidx_v, r0b, r1b, r2b, r3b,
                    i0, i1, i2, i3, g0, g1, g2, g3, s0, s1, s2, s3,
                    acc_sh):
        rows = [r0b, r1b, r2b, r3b]
        semi = [i0, i1, i2, i3]
        semg = [g0, g1, g2, g3]
        sems = [s0, s1, s2, s3]
        c = lax.axis_index("c")
        s = lax.axis_index("s")
        wid = c * NS + s
        rb = s * _AGG_RPS
        pltpu.sync_copy(zeros_hbm, acc_sh.at[pl.ds(rb, _AGG_RPS)])
        pltpu.sync_copy(sidx_hbm.at[wid], sidx_v)

        def gi_start(w, b):
            pltpu.make_async_copy(gidx_hbm.at[wid, w], gidx_v.at[b],
                                  semi[b]).start()

        def gi_wait(w, b):
            pltpu.make_async_copy(gidx_hbm.at[wid, w], gidx_v.at[b],
                                  semi[b]).wait()

        def g_start(b):
            pltpu.make_async_copy(y_hbm.at[gidx_v.at[b]], rows[b],
                                  semg[b]).start()

        def g_wait(b):
            pltpu.make_async_copy(y_hbm.at[gidx_v.at[b]], rows[b],
                                  semg[b]).wait()

        def s_start(w, b):
            pltpu.async_copy(rows[b], acc_sh.at[sidx_v.at[w]], sems[b],
                             add=True)

        def s_wait(w, b):
            pltpu.make_async_copy(rows[b], acc_sh.at[sidx_v.at[w]],
                                  sems[b]).wait()

        plsc.subcore_barrier()

        for b in range(4):
            gi_start(b, b)
        for b in range(2):
            gi_wait(b, b)
            g_start(b)

        @pl.loop(0, NWIN, step=4)
        def _(j):
            for b in range(4):
                w = j + b

                @pl.when(w >= 2)
                def _():
                    s_wait(w - 2, (b + 2) % 4)

                g_wait(b)

                @pl.when(w + 4 < NWIN)
                def _():
                    gi_start(w + 4, b)

                @pl.when(w + 2 < NWIN)
                def _():
                    gi_wait(w + 2, (b + 2) % 4)
                    g_start((b + 2) % 4)

                s_start(w, b)

        s_wait(NWIN - 2, (NWIN - 2) % 4)
        s_wait(NWIN - 1, (NWIN - 1) % 4)
        plsc.subcore_barrier()
        pltpu.sync_copy(acc_sh.at[pl.ds(rb, _AGG_RPS)],
                        out_hbm.at[c, pl.ds(rb, _AGG_RPS)])

    return _agg_kernel


_W3COLS = 16
_agg128 = _make_agg_kernel(D, jnp.bfloat16)
_agg3 = _agg128 if _W3COLS == D else _make_agg_kernel_deep(_W3COLS)


def _prep_body(degp_ref, feat_ref, ns_ref, nd_ref, y1_ref):
    od = degp_ref[0, :N, 0:1] + degp_ref[1, :N, 0:1]
    idg = (degp_ref[0, NROWS:NROWS + N, 0:1]
           + degp_ref[1, NROWS:NROWS + N, 0:1])
    ns = jnp.where(od > 0, lax.rsqrt(od), 0.0)
    nd = jnp.where(idg > 0, lax.rsqrt(idg), 0.0)
    ns_ref[...] = ns
    nd_ref[...] = nd
    y1_ref[...] = (feat_ref[...] * ns).astype(y1_ref.dtype)


def _f1_body(feat_ref, W_ref, o_ref):
    o_ref[...] = jnp.dot(feat_ref[...].astype(jnp.bfloat16),
                         W_ref[...].astype(jnp.bfloat16),
                         preferred_element_type=jnp.float32)


def _mid_body(p_ref, nd_ref, b_ref, g_ref, be_ref, ns_ref, Wn_ref, o_ref):
    agg = (p_ref[0, :N, :].astype(jnp.float32)
           + p_ref[1, :N, :].astype(jnp.float32))
    t = agg * nd_ref[...] + b_ref[...]
    mu = jnp.mean(t, -1, keepdims=True)
    var = jnp.mean((t - mu) ** 2, -1, keepdims=True)
    x = (t - mu) / jnp.sqrt(var + 1e-5) * g_ref[...] + be_ref[...]
    x = jnp.where(x > 0, x, 0.01 * x)
    y = jnp.dot(x.astype(jnp.bfloat16), Wn_ref[...].astype(jnp.bfloat16),
                preferred_element_type=jnp.float32) * ns_ref[...]
    o_ref[...] = y.astype(o_ref.dtype)


def _fin_body(p_ref, nd_ref, b3_ref, o_ref):
    a = p_ref[0, :N, :O] + p_ref[1, :N, :O]
    o_ref[...] = a * nd_ref[...] + b3_ref[...]


def kernel(features, edge_index, W1, b1, g1, beta1, W2, b2, g2, beta2, W3, b3):
    src = edge_index[0]
    dst = edge_index[1]

    pad = EP - E
    k = jnp.arange(pad, dtype=jnp.int32)
    kp = k % NPAD
    trash = N + kp
    src_g = jnp.concatenate([src, kp]).reshape(NW, NWIN, WIN)
    dst_s = jnp.concatenate([dst, trash]).reshape(NW, NWIN, WIN)
    src_d = jnp.concatenate([src, trash]).reshape(NW, NWIN, WIN)
    dst_d = jnp.concatenate([dst + NROWS, trash + NROWS]).reshape(NW, NWIN, WIN)

    ones16 = jnp.ones((WIN, 16), jnp.float32)
    zdeg = jnp.zeros((_DEG_RPS, 16), jnp.float32)
    zagg = jnp.zeros((_AGG_RPS, D), jnp.bfloat16)
    zagg16 = jnp.zeros((_AGG_RPS, _W3COLS), jnp.float32)

    F1 = pl.pallas_call(
        _f1_body,
        out_shape=jax.ShapeDtypeStruct((N, H), jnp.float32),
    )(features, W1)
    degp = _deg_kernel(src_d, dst_d, ones16, zdeg)

    ns, nd, y1 = pl.pallas_call(
        _prep_body,
        out_shape=(jax.ShapeDtypeStruct((N, 1), jnp.float32),
                   jax.ShapeDtypeStruct((N, 1), jnp.float32),
                   jax.ShapeDtypeStruct((N, H), jnp.bfloat16)),
    )(degp, F1)

    b1r, g1r, be1r = b1.reshape(1, H), g1.reshape(1, H), beta1.reshape(1, H)
    b2r, g2r, be2r = b2.reshape(1, H), g2.reshape(1, H), beta2.reshape(1, H)
    W3p = jnp.pad(W3, ((0, 0), (0, _W3COLS - O)))
    b3r = b3.reshape(1, O)

    p1 = _agg128(y1, src_g, dst_s, zagg)
    y2 = pl.pallas_call(
        _mid_body,
        out_shape=jax.ShapeDtypeStruct((N, H), jnp.bfloat16),
    )(p1, nd, b1r, g1r, be1r, ns, W2)

    p2 = _agg128(y2, src_g, dst_s, zagg)
    z3 = pl.pallas_call(
        _mid_body,
        out_shape=jax.ShapeDtypeStruct((N, _W3COLS), jnp.float32),
    )(p2, nd, b2r, g2r, be2r, ns, W3p)

    p3 = _agg3(z3, src_g, dst_s, zagg16)
    out = pl.pallas_call(
        _fin_body,
        out_shape=jax.ShapeDtypeStruct((N, O), jnp.float32),
    )(p3, nd, b3r)
    return out


# speedup vs baseline: 12.3225x; 1.0813x over previous
import functools

import jax
import jax.numpy as jnp
from jax import lax
from jax.experimental import pallas as pl
from jax.experimental.pallas import tpu as pltpu
from jax.experimental.pallas import tpu_sc as plsc

N, E, D, H, O = 10000, 320000, 128, 128, 2
NC, NS = 2, 16
NW = NC * NS
WIN = 128
NPAD = 16
NROWS = 10112
NWIN = 2 * (-(-(-(-E // NW)) // (2 * WIN)))
EPW = NWIN * WIN
EP = EPW * NW
DEG_ROWS = 2 * NROWS
_DEG_RPS = DEG_ROWS // NS
_AGG_RPS = NROWS // NS

_mesh = plsc.VectorSubcoreMesh(core_axis_name="c", subcore_axis_name="s")


@functools.partial(
    pl.kernel,
    mesh=_mesh,
    out_type=jax.ShapeDtypeStruct((NC, DEG_ROWS, 16), jnp.float32),
    compiler_params=pltpu.CompilerParams(use_tc_tiling_on_sc=False),
    scratch_types=[
        pltpu.VMEM((NWIN, WIN), jnp.int32),
        pltpu.VMEM((NWIN, WIN), jnp.int32),
        pltpu.VMEM((WIN, 16), jnp.float32),
        pltpu.SemaphoreType.DMA,
        pltpu.VMEM_SHARED((DEG_ROWS, 16), jnp.float32),
    ],
)
def _deg_kernel(sidx_hbm, didx_hbm, ones_hbm, zeros_hbm, out_hbm,
                sidx_v, didx_v, ones_v, dsem, acc_sh):
    c = lax.axis_index("c")
    s = lax.axis_index("s")
    wid = c * NS + s
    r0 = s * _DEG_RPS
    pltpu.sync_copy(zeros_hbm, acc_sh.at[pl.ds(r0, _DEG_RPS)])
    pltpu.sync_copy(ones_hbm, ones_v)
    pltpu.sync_copy(sidx_hbm.at[wid], sidx_v)
    pltpu.sync_copy(didx_hbm.at[wid], didx_v)
    plsc.subcore_barrier()

    @pl.loop(0, NWIN)
    def _(j):
        pltpu.async_copy(ones_v, acc_sh.at[sidx_v.at[j]], dsem, add=True)
        pltpu.async_copy(ones_v, acc_sh.at[didx_v.at[j]], dsem, add=True)

        @pl.when(j > 0)
        def _():
            pltpu.make_async_copy(ones_v, acc_sh.at[sidx_v.at[j]],
                                  dsem).wait()
            pltpu.make_async_copy(ones_v, acc_sh.at[didx_v.at[j]],
                                  dsem).wait()

    pltpu.make_async_copy(ones_v, acc_sh.at[sidx_v.at[0]], dsem).wait()
    pltpu.make_async_copy(ones_v, acc_sh.at[didx_v.at[0]], dsem).wait()
    plsc.subcore_barrier()
    pltpu.sync_copy(acc_sh.at[pl.ds(r0, _DEG_RPS)],
                    out_hbm.at[c, pl.ds(r0, _DEG_RPS)])


def _make_agg_kernel_resident(width, dtype=jnp.float32):
    @functools.partial(
        pl.kernel,
        mesh=_mesh,
        out_type=jax.ShapeDtypeStruct((NC, NROWS, width), dtype),
        compiler_params=pltpu.CompilerParams(use_tc_tiling_on_sc=False),
        scratch_types=[
            pltpu.VMEM((2, WIN), jnp.int32),
            pltpu.VMEM((NWIN, WIN), jnp.int32),
            pltpu.VMEM((WIN, width), dtype),
            pltpu.VMEM((WIN, width), dtype),
            pltpu.SemaphoreType.DMA,
            pltpu.SemaphoreType.DMA,
            pltpu.SemaphoreType.DMA,
            pltpu.SemaphoreType.DMA,
            pltpu.SemaphoreType.DMA,
            pltpu.SemaphoreType.DMA,
            pltpu.VMEM_SHARED((NROWS, width), dtype),
            pltpu.VMEM_SHARED((NROWS, width), dtype),
        ],
    )
    def _agg_kernel(y_hbm, gidx_hbm, sidx_hbm, zeros_hbm, out_hbm,
                    gidx_v, sidx_v, rows0, rows1, semi0, semi1, semg0, semg1,
                    sems0, sems1, acc_sh, tab_sh):
        c = lax.axis_index("c")
        s = lax.axis_index("s")
        wid = c * NS + s
        r0 = s * _AGG_RPS
        pltpu.sync_copy(zeros_hbm, acc_sh.at[pl.ds(r0, _AGG_RPS)])
        pltpu.sync_copy(y_hbm.at[pl.ds(r0, _AGG_RPS)],
                        tab_sh.at[pl.ds(r0, _AGG_RPS)])
        pltpu.sync_copy(sidx_hbm.at[wid], sidx_v)

        def gidx_start(j, b, sem):
            pltpu.make_async_copy(gidx_hbm.at[wid, j], gidx_v.at[b],
                                  sem).start()

        def gidx_wait(j, b, sem):
            pltpu.make_async_copy(gidx_hbm.at[wid, j], gidx_v.at[b],
                                  sem).wait()

        plsc.subcore_barrier()

        gidx_start(0, 0, semi0)
        gidx_wait(0, 0, semi0)
        pltpu.make_async_copy(tab_sh.at[gidx_v.at[0]], rows0, semg0).start()
        gidx_start(1, 1, semi1)

        @pl.loop(0, NWIN, step=2)
        def _(j):
            @pl.when(j > 0)
            def _():
                pltpu.make_async_copy(rows1, acc_sh.at[sidx_v.at[j]],
                                      sems1).wait()

            pltpu.make_async_copy(tab_sh.at[gidx_v.at[0]], rows0,
                                  semg0).wait()
            gidx_wait(j + 1, 1, semi1)
            pltpu.make_async_copy(tab_sh.at[gidx_v.at[1]], rows1,
                                  semg1).start()
            pltpu.async_copy(rows0, acc_sh.at[sidx_v.at[j]], sems0, add=True)

            @pl.when(j + 2 < NWIN)
            def _():
                gidx_start(j + 2, 0, semi0)

            pltpu.make_async_copy(tab_sh.at[gidx_v.at[1]], rows1,
                                  semg1).wait()
            pltpu.make_async_copy(rows0, acc_sh.at[sidx_v.at[j]],
                                  sems0).wait()

            @pl.when(j + 2 < NWIN)
            def _():
                gidx_wait(j + 2, 0, semi0)
                pltpu.make_async_copy(tab_sh.at[gidx_v.at[0]], rows0,
                                      semg0).start()

            pltpu.async_copy(rows1, acc_sh.at[sidx_v.at[j + 1]], sems1,
                             add=True)

            @pl.when(j + 3 < NWIN)
            def _():
                gidx_start(j + 3, 1, semi1)

        pltpu.make_async_copy(rows1, acc_sh.at[sidx_v.at[NWIN - 1]],
                              sems1).wait()
        plsc.subcore_barrier()
        pltpu.sync_copy(acc_sh.at[pl.ds(r0, _AGG_RPS)],
                        out_hbm.at[c, pl.ds(r0, _AGG_RPS)])

    return _agg_kernel


def _make_agg_kernel(width, dtype=jnp.float32):
    @functools.partial(
        pl.kernel,
        mesh=_mesh,
        out_type=jax.ShapeDtypeStruct((NC, NROWS, width), dtype),
        compiler_params=pltpu.CompilerParams(use_tc_tiling_on_sc=False),
        scratch_types=[
            pltpu.VMEM((2, WIN), jnp.int32),
            pltpu.VMEM((NWIN, WIN), jnp.int32),
            pltpu.VMEM((WIN, width), dtype),
            pltpu.VMEM((WIN, width), dtype),
            pltpu.SemaphoreType.DMA,
            pltpu.SemaphoreType.DMA,
            pltpu.SemaphoreType.DMA,
            pltpu.SemaphoreType.DMA,
            pltpu.SemaphoreType.DMA,
            pltpu.SemaphoreType.DMA,
            pltpu.VMEM_SHARED((NROWS, width), dtype),
        ],
    )
    def _agg_kernel(y_hbm, gidx_hbm, sidx_hbm, zeros_hbm, out_hbm,
                    gidx_v, sidx_v, rows0, rows1, semi0, semi1, semg0, semg1,
                    sems0, sems1, acc_sh):
        c = lax.axis_index("c")
        s = lax.axis_index("s")
        wid = c * NS + s
        r0 = s * _AGG_RPS
        pltpu.sync_copy(zeros_hbm, acc_sh.at[pl.ds(r0, _AGG_RPS)])
        pltpu.sync_copy(sidx_hbm.at[wid], sidx_v)

        def gidx_start(j, b, sem):
            pltpu.make_async_copy(gidx_hbm.at[wid, j], gidx_v.at[b],
                                  sem).start()

        def gidx_wait(j, b, sem):
            pltpu.make_async_copy(gidx_hbm.at[wid, j], gidx_v.at[b],
                                  sem).wait()

        plsc.subcore_barrier()

        gidx_start(0, 0, semi0)
        gidx_wait(0, 0, semi0)
        pltpu.make_async_copy(y_hbm.at[gidx_v.at[0]], rows0, semg0).start()
        gidx_start(1, 1, semi1)

        @pl.loop(0, NWIN, step=2)
        def _(j):
            @pl.when(j > 0)
            def _():
                pltpu.make_async_copy(rows1, acc_sh.at[sidx_v.at[j]],
                                      sems1).wait()

            pltpu.make_async_copy(y_hbm.at[gidx_v.at[0]], rows0,
                                  semg0).wait()
            gidx_wait(j + 1, 1, semi1)
            pltpu.make_async_copy(y_hbm.at[gidx_v.at[1]], rows1,
                                  semg1).start()
            pltpu.async_copy(rows0, acc_sh.at[sidx_v.at[j]], sems0, add=True)

            @pl.when(j + 2 < NWIN)
            def _():
                gidx_start(j + 2, 0, semi0)

            pltpu.make_async_copy(y_hbm.at[gidx_v.at[1]], rows1,
                                  semg1).wait()
            pltpu.make_async_copy(rows0, acc_sh.at[sidx_v.at[j]],
                                  sems0).wait()

            @pl.when(j + 2 < NWIN)
            def _():
                gidx_wait(j + 2, 0, semi0)
                pltpu.make_async_copy(y_hbm.at[gidx_v.at[0]], rows0,
                                      semg0).start()

            pltpu.async_copy(rows1, acc_sh.at[sidx_v.at[j + 1]], sems1,
                             add=True)

            @pl.when(j + 3 < NWIN)
            def _():
                gidx_start(j + 3, 1, semi1)

        pltpu.make_async_copy(rows1, acc_sh.at[sidx_v.at[NWIN - 1]],
                              sems1).wait()
        plsc.subcore_barrier()
        pltpu.sync_copy(acc_sh.at[pl.ds(r0, _AGG_RPS)],
                        out_hbm.at[c, pl.ds(r0, _AGG_RPS)])

    return _agg_kernel


def _make_agg_kernel_deep(width):
    @functools.partial(
        pl.kernel,
        mesh=_mesh,
        out_type=jax.ShapeDtypeStruct((NC, NROWS, width), jnp.float32),
        compiler_params=pltpu.CompilerParams(use_tc_tiling_on_sc=False),
        scratch_types=(
            [pltpu.VMEM((4, WIN), jnp.int32),
             pltpu.VMEM((NWIN, WIN), jnp.int32)]
            + [pltpu.VMEM((WIN, width), jnp.float32) for _ in range(4)]
            + [pltpu.SemaphoreType.DMA for _ in range(12)]
            + [pltpu.VMEM_SHARED((NROWS, width), jnp.float32)]
        ),
    )
    def _agg_kernel(y_hbm, gidx_hbm, sidx_hbm, zeros_hbm, out_hbm,
                    gidx_v, sidx_v, r0b, r1b, r2b, r3b,
                    i0, i1, i2, i3, g0, g1, g2, g3, s0, s1, s2, s3,
                    acc_sh):
        rows = [r0b, r1b, r2b, r3b]
        semi = [i0, i1, i2, i3]
        semg = [g0, g1, g2, g3]
        sems = [s0, s1, s2, s3]
        c = lax.axis_index("c")
        s = lax.axis_index("s")
        wid = c * NS + s
        rb = s * _AGG_RPS
        pltpu.sync_copy(zeros_hbm, acc_sh.at[pl.ds(rb, _AGG_RPS)])
        pltpu.sync_copy(sidx_hbm.at[wid], sidx_v)

        def gi_start(w, b):
            pltpu.make_async_copy(gidx_hbm.at[wid, w], gidx_v.at[b],
                                  semi[b]).start()

        def gi_wait(w, b):
            pltpu.make_async_copy(gidx_hbm.at[wid, w], gidx_v.at[b],
                                  semi[b]).wait()

        def g_start(b):
            pltpu.make_async_copy(y_hbm.at[gidx_v.at[b]], rows[b],
                                  semg[b]).start()

        def g_wait(b):
            pltpu.make_async_copy(y_hbm.at[gidx_v.at[b]], rows[b],
                                  semg[b]).wait()

        def s_start(w, b):
            pltpu.async_copy(rows[b], acc_sh.at[sidx_v.at[w]], sems[b],
                             add=True)

        def s_wait(w, b):
            pltpu.make_async_copy(rows[b], acc_sh.at[sidx_v.at[w]],
                                  sems[b]).wait()

        plsc.subcore_barrier()

        for b in range(4):
            gi_start(b, b)
        for b in range(2):
            gi_wait(b, b)
            g_start(b)

        @pl.loop(0, NWIN, step=4)
        def _(j):
            for b in range(4):
                w = j + b

                @pl.when(w >= 2)
                def _():
                    s_wait(w - 2, (b + 2) % 4)

                g_wait(b)

                @pl.when(w + 4 < NWIN)
                def _():
                    gi_start(w + 4, b)

                @pl.when(w + 2 < NWIN)
                def _():
                    gi_wait(w + 2, (b + 2) % 4)
                    g_start((b + 2) % 4)

                s_start(w, b)

        s_wait(NWIN - 2, (NWIN - 2) % 4)
        s_wait(NWIN - 1, (NWIN - 1) % 4)
        plsc.subcore_barrier()
        pltpu.sync_copy(acc_sh.at[pl.ds(rb, _AGG_RPS)],
                        out_hbm.at[c, pl.ds(rb, _AGG_RPS)])

    return _agg_kernel


_W3COLS = 16
_agg128 = _make_agg_kernel_resident(D, jnp.bfloat16)
_agg3 = _agg128 if _W3COLS == D else _make_agg_kernel_deep(_W3COLS)


def _prep_body(degp_ref, feat_ref, ns_ref, nd_ref, y1_ref):
    od = degp_ref[0, :N, 0:1] + degp_ref[1, :N, 0:1]
    idg = (degp_ref[0, NROWS:NROWS + N, 0:1]
           + degp_ref[1, NROWS:NROWS + N, 0:1])
    ns = jnp.where(od > 0, lax.rsqrt(od), 0.0)
    nd = jnp.where(idg > 0, lax.rsqrt(idg), 0.0)
    ns_ref[...] = ns
    nd_ref[...] = nd
    y1_ref[...] = (feat_ref[...] * ns).astype(y1_ref.dtype)


def _f1_body(feat_ref, W_ref, o_ref):
    o_ref[...] = jnp.dot(feat_ref[...].astype(jnp.bfloat16),
                         W_ref[...].astype(jnp.bfloat16),
                         preferred_element_type=jnp.float32)


def _mid_body(p_ref, nd_ref, b_ref, g_ref, be_ref, ns_ref, Wn_ref, o_ref):
    agg = (p_ref[0, :N, :].astype(jnp.float32)
           + p_ref[1, :N, :].astype(jnp.float32))
    t = agg * nd_ref[...] + b_ref[...]
    mu = jnp.mean(t, -1, keepdims=True)
    var = jnp.mean((t - mu) ** 2, -1, keepdims=True)
    x = (t - mu) / jnp.sqrt(var + 1e-5) * g_ref[...] + be_ref[...]
    x = jnp.where(x > 0, x, 0.01 * x)
    y = jnp.dot(x.astype(jnp.bfloat16), Wn_ref[...].astype(jnp.bfloat16),
                preferred_element_type=jnp.float32) * ns_ref[...]
    o_ref[...] = y.astype(o_ref.dtype)


def _fin_body(p_ref, nd_ref, b3_ref, o_ref):
    a = p_ref[0, :N, :O] + p_ref[1, :N, :O]
    o_ref[...] = a * nd_ref[...] + b3_ref[...]


def kernel(features, edge_index, W1, b1, g1, beta1, W2, b2, g2, beta2, W3, b3):
    src = edge_index[0]
    dst = edge_index[1]

    pad = EP - E
    k = jnp.arange(pad, dtype=jnp.int32)
    kp = k % NPAD
    trash = N + kp
    src_g = jnp.concatenate([src, kp]).reshape(NW, NWIN, WIN)
    dst_s = jnp.concatenate([dst, trash]).reshape(NW, NWIN, WIN)
    src_d = jnp.concatenate([src, trash]).reshape(NW, NWIN, WIN)
    dst_d = jnp.concatenate([dst + NROWS, trash + NROWS]).reshape(NW, NWIN, WIN)

    ones16 = jnp.ones((WIN, 16), jnp.float32)
    zdeg = jnp.zeros((_DEG_RPS, 16), jnp.float32)
    zagg = jnp.zeros((_AGG_RPS, D), jnp.bfloat16)
    zagg16 = jnp.zeros((_AGG_RPS, _W3COLS), jnp.float32)

    F1 = pl.pallas_call(
        _f1_body,
        out_shape=jax.ShapeDtypeStruct((N, H), jnp.float32),
    )(features, W1)
    degp = _deg_kernel(src_d, dst_d, ones16, zdeg)

    ns, nd, y1 = pl.pallas_call(
        _prep_body,
        out_shape=(jax.ShapeDtypeStruct((N, 1), jnp.float32),
                   jax.ShapeDtypeStruct((N, 1), jnp.float32),
                   jax.ShapeDtypeStruct((N, H), jnp.bfloat16)),
    )(degp, F1)

    b1r, g1r, be1r = b1.reshape(1, H), g1.reshape(1, H), beta1.reshape(1, H)
    b2r, g2r, be2r = b2.reshape(1, H), g2.reshape(1, H), beta2.reshape(1, H)
    W3p = jnp.pad(W3, ((0, 0), (0, _W3COLS - O)))
    b3r = b3.reshape(1, O)

    p1 = _agg128(jnp.pad(y1, ((0, NROWS - N), (0, 0))), src_g, dst_s, zagg)
    y2 = pl.pallas_call(
        _mid_body,
        out_shape=jax.ShapeDtypeStruct((N, H), jnp.bfloat16),
    )(p1, nd, b1r, g1r, be1r, ns, W2)

    p2 = _agg128(jnp.pad(y2, ((0, NROWS - N), (0, 0))), src_g, dst_s, zagg)
    z3 = pl.pallas_call(
        _mid_body,
        out_shape=jax.ShapeDtypeStruct((N, _W3COLS), jnp.float32),
    )(p2, nd, b2r, g2r, be2r, ns, W3p)

    p3 = _agg3(z3, src_g, dst_s, zagg16)
    out = pl.pallas_call(
        _fin_body,
        out_shape=jax.ShapeDtypeStruct((N, O), jnp.float32),
    )(p3, nd, b3r)
    return out
